# baseline (speedup 1.0000x reference)
import jax
import jax.numpy as jnp
from jax.experimental import pallas as pl

TOKENS = 16384
HIDDEN = 4096
EXPERTS = 64

BM = 512
_KS = 8

_DN = (((1,), (1,)), ((), ()))


def _router_body(x_ref, w1_ref, b1_ref, w2_ref, b2_ref, o_ref):
    ksz = HIDDEN // _KS
    h = None
    for k in range(_KS):
        xk = x_ref[:, pl.ds(k * ksz, ksz)].astype(jnp.bfloat16)
        hk = jax.lax.dot_general(
            xk, w1_ref[:, pl.ds(k * ksz, ksz)], _DN,
            preferred_element_type=jnp.float32)
        h = hk if h is None else h + hk
    h = jnp.maximum(h + b1_ref[...], 0.0)
    p = jax.lax.dot_general(h, w2_ref[...], _DN,
                            preferred_element_type=jnp.float32)
    o_ref[...] = p + b2_ref[...]


def kernel(x, W1, b1, W2, b2):
    w1b = W1.astype(jnp.bfloat16)
    b1r = b1.reshape(1, HIDDEN)
    b2r = b2.reshape(1, EXPERTS)

    grid = (TOKENS // BM,)
    return pl.pallas_call(
        _router_body,
        grid=grid,
        in_specs=[
            pl.BlockSpec((BM, HIDDEN), lambda m: (m, 0)),
            pl.BlockSpec((HIDDEN, HIDDEN), lambda m: (0, 0)),
            pl.BlockSpec((1, HIDDEN), lambda m: (0, 0)),
            pl.BlockSpec((EXPERTS, HIDDEN), lambda m: (0, 0)),
            pl.BlockSpec((1, EXPERTS), lambda m: (0, 0)),
        ],
        out_specs=pl.BlockSpec((BM, EXPERTS), lambda m: (m, 0)),
        out_shape=jax.ShapeDtypeStruct((TOKENS, EXPERTS), jnp.float32),
    )(x, w1b, b1r, W2, b2r)


# speedup vs baseline: 1.0069x; 1.0069x over previous
import jax
import jax.numpy as jnp
from jax.experimental import pallas as pl
from jax.experimental.pallas import tpu as pltpu

TOKENS = 16384
HIDDEN = 4096
EXPERTS = 64

BM = 512
BN = 4096
NCHUNK = HIDDEN // BN

_DN = (((1,), (1,)), ((), ()))


_KS = 8


def _router_body(x_ref, w1_ref, b1_ref, w2_ref, b2_ref, o_ref, xb_ref):
    acc = jnp.broadcast_to(b2_ref[...], (BM, EXPERTS))

    ksz = HIDDEN // _KS
    h = None
    for k in range(_KS):
        xk = x_ref[:, pl.ds(k * ksz, ksz)].astype(jnp.bfloat16)
        xb_ref[:, pl.ds(k * ksz, ksz)] = xk
        hk = jax.lax.dot_general(
            xk, w1_ref[pl.ds(0, BN), pl.ds(k * ksz, ksz)], _DN,
            preferred_element_type=jnp.float32)
        h = hk if h is None else h + hk
    h = jnp.maximum(h + b1_ref[:, pl.ds(0, BN)], 0.0)
    acc = acc + jax.lax.dot_general(h, w2_ref[:, pl.ds(0, BN)], _DN,
                                    preferred_element_type=jnp.float32)

    xb = xb_ref[...]
    for j in range(1, NCHUNK):
        w1c = w1_ref[pl.ds(j * BN, BN), :]
        h = jax.lax.dot_general(xb, w1c, _DN,
                                preferred_element_type=jnp.float32)
        h = jnp.maximum(h + b1_ref[:, pl.ds(j * BN, BN)], 0.0)
        w2c = w2_ref[:, pl.ds(j * BN, BN)]
        acc = acc + jax.lax.dot_general(h, w2c, _DN,
                                        preferred_element_type=jnp.float32)
    o_ref[...] = acc


def kernel(x, W1, b1, W2, b2):
    w1b = W1.astype(jnp.bfloat16)
    b1r = b1.reshape(1, HIDDEN)
    b2r = b2.reshape(1, EXPERTS)

    grid = (TOKENS // BM,)
    return pl.pallas_call(
        _router_body,
        grid=grid,
        in_specs=[
            pl.BlockSpec((BM, HIDDEN), lambda m: (m, 0)),
            pl.BlockSpec((HIDDEN, HIDDEN), lambda m: (0, 0)),
            pl.BlockSpec((1, HIDDEN), lambda m: (0, 0)),
            pl.BlockSpec((EXPERTS, HIDDEN), lambda m: (0, 0)),
            pl.BlockSpec((1, EXPERTS), lambda m: (0, 0)),
        ],
        out_specs=pl.BlockSpec((BM, EXPERTS), lambda m: (m, 0)),
        out_shape=jax.ShapeDtypeStruct((TOKENS, EXPERTS), jnp.float32),
        scratch_shapes=[pltpu.VMEM((BM, HIDDEN), jnp.bfloat16)],
    )(x, w1b, b1r, W2, b2r)
